# baseline (speedup 1.0000x reference)
import functools

import jax
import jax.numpy as jnp
from jax import lax
from jax.experimental import pallas as pl
from jax.experimental.pallas import tpu as pltpu
from jax.experimental.pallas import tpu_sc as plsc

_N = 10000
_E = 320000
_D = 128
_NC = 2
_NS = 16
_W = 16
_NCHUNK = 1256
_EPT = _E // _NS
_PAD = _NCHUNK * _W - _EPT
_RPT = 640
_ZB = 16


def _edge_phase(t2, src3, dst3):
  mesh = plsc.VectorSubcoreMesh(core_axis_name="c", subcore_axis_name="s")

  @functools.partial(
      pl.kernel,
      out_type=jax.ShapeDtypeStruct((_NC, _N, 2 * 64), jnp.float32),
      mesh=mesh,
      scratch_types=[
          pltpu.VMEM((8, _W), jnp.int32),
          pltpu.VMEM((8, _W), jnp.int32),
          pltpu.VMEM((_W, 128), jnp.float32),
          pltpu.VMEM((_W, 128), jnp.float32),
          pltpu.VMEM((_W, 128), jnp.float32),
          pltpu.VMEM((_W, 128), jnp.float32),
          pltpu.VMEM_SHARED((_N + 8, 128), jnp.float32),
          pltpu.SemaphoreType.DMA,
          pltpu.SemaphoreType.DMA,
          pltpu.SemaphoreType.DMA,
          pltpu.SemaphoreType.DMA,
          pltpu.SemaphoreType.DMA,
          pltpu.SemaphoreType.DMA,
          pltpu.SemaphoreType.DMA,
          pltpu.SemaphoreType.DMA,
          pltpu.SemaphoreType.DMA,
          pltpu.SemaphoreType.DMA,
          pltpu.SemaphoreType.DMA,
          pltpu.SemaphoreType.DMA,
          pltpu.SemaphoreType.DMA,
          pltpu.SemaphoreType.DMA,
          pltpu.SemaphoreType.DMA,
          pltpu.SemaphoreType.DMA,
      ],
  )
  def k(t2_hbm, src_hbm, dst_hbm, acc_hbm, isrc, idst, buf0, buf1, buf2,
        buf3, acc_sh, gsem0, gsem1, gsem2, gsem3, ssem0, ssem1, ssem2,
        ssem3, isem0, isem1, isem2, isem3, isem4, isem5, isem6, isem7):
    c = lax.axis_index("c")
    s = lax.axis_index("s")
    zeros16 = jnp.zeros((16,), jnp.float32)
    buf = (buf0, buf1, buf2, buf3)
    gsem = (gsem0, gsem1, gsem2, gsem3)
    ssem = (ssem0, ssem1, ssem2, ssem3)
    isem = (isem0, isem1, isem2, isem3, isem4, isem5, isem6, isem7)
    coff = c * _N

    def idx_start(q, slot):
      pltpu.async_copy(src_hbm.at[s].at[q], isrc.at[slot], isem[slot])
      pltpu.async_copy(dst_hbm.at[s].at[q], idst.at[slot], isem[slot])

    def idx_wait_off(slot):
      pltpu.make_async_copy(src_hbm.at[s].at[0], isrc.at[slot],
                            isem[slot]).wait()
      pltpu.make_async_copy(dst_hbm.at[s].at[0], idst.at[slot],
                            isem[slot]).wait()
      for i in range(_W // 16):
        isrc[slot, pl.ds(i * 16, 16)] = isrc[slot, pl.ds(i * 16, 16)] + coff

    for q0 in range(4):
      idx_start(q0, q0)

    @pl.loop(0, _W)
    def _(e):
      for g in range(8):
        buf1[e, pl.ds(g * 16, 16)] = zeros16
        buf2[e, pl.ds(g * 16, 16)] = zeros16
        buf3[e, pl.ds(g * 16, 16)] = zeros16

    base_row = s * _RPT
    nrep = jnp.where(s == _NS - 1, (_N - (_NS - 1) * _RPT) // _ZB, _RPT // _ZB)

    @pl.loop(0, nrep)
    def _(j):
      pltpu.sync_copy(buf2.at[pl.ds(0, _ZB)],
                      acc_sh.at[pl.ds(base_row + j * _ZB, _ZB)])

    plsc.subcore_barrier()

    idx_wait_off(0)
    pltpu.async_copy(buf1, acc_sh.at[idst.at[0]], ssem1, add=True)
    pltpu.async_copy(buf2, acc_sh.at[idst.at[0]], ssem2, add=True)
    pltpu.async_copy(buf3, acc_sh.at[idst.at[0]], ssem3, add=True)
    pltpu.async_copy(t2_hbm.at[isrc.at[0]], buf0, gsem0)

    def chunk_body(q, j, gather_next, idx_next):
      p = j % 4
      pn = (j + 1) % 4
      if gather_next:
        pltpu.make_async_copy(buf[pn], acc_sh.at[idst.at[j]],
                              ssem[pn]).wait()
        idx_wait_off((j + 1) % 8)
        pltpu.async_copy(t2_hbm.at[isrc.at[(j + 1) % 8]], buf[pn], gsem[pn])
      if idx_next:
        idx_start(q + 4, (j + 4) % 8)
      pltpu.make_async_copy(t2_hbm.at[isrc.at[j]], buf[p], gsem[p]).wait()
      pltpu.async_copy(buf[p], acc_sh.at[idst.at[j]], ssem[p], add=True)

    @pl.loop(0, _NCHUNK - 8, step=8)
    def _(kk):
      for j in range(8):
        chunk_body(kk + j, j, True, True)

    for j in range(8):
      q = _NCHUNK - 8 + j
      chunk_body(q, j, q + 1 < _NCHUNK, q + 4 < _NCHUNK)

    pltpu.make_async_copy(buf0, acc_sh.at[idst.at[0]], ssem0).wait()
    pltpu.make_async_copy(buf1, acc_sh.at[idst.at[1]], ssem1).wait()
    pltpu.make_async_copy(buf2, acc_sh.at[idst.at[2]], ssem2).wait()
    pltpu.make_async_copy(buf3, acc_sh.at[idst.at[3]], ssem3).wait()
    plsc.subcore_barrier()

    nrep2 = jnp.where(s == _NS - 1, (_N - (_NS - 1) * _RPT) // _ZB,
                      _RPT // _ZB)

    @pl.loop(0, nrep2)
    def _(j):
      r0 = base_row + j * _ZB
      pltpu.sync_copy(acc_sh.at[pl.ds(r0, _ZB)],
                      acc_hbm.at[c].at[pl.ds(r0, _ZB)])

  return k(t2, src3, dst3)


_BLK = 400


def _encode(x, w, b):
  def body(x_ref, w_ref, b_ref, o_ref):
    o_ref[...] = (
        jnp.dot(x_ref[...], w_ref[...], preferred_element_type=jnp.float32)
        + b_ref[...])

  return pl.pallas_call(
      body,
      grid=(_N // _BLK,),
      in_specs=[
          pl.BlockSpec((_BLK, _D), lambda i: (i, 0)),
          pl.BlockSpec((_D, _D), lambda i: (0, 0)),
          pl.BlockSpec((1, _D), lambda i: (0, 0)),
      ],
      out_specs=pl.BlockSpec((_BLK, _D), lambda i: (i, 0)),
      out_shape=jax.ShapeDtypeStruct((_N, _D), jnp.float32),
  )(x, w, b.reshape(1, _D))


def _ln_table(h, g, b):
  def body(h_ref, g_ref, b_ref, hh_ref, t2_ref):
    hv = h_ref[...]
    mu = jnp.mean(hv, axis=1, keepdims=True)
    d = hv - mu
    var = jnp.mean(d * d, axis=1, keepdims=True)
    y = g_ref[...] * d / jnp.sqrt(var + 1e-5) + b_ref[...]
    y = jnp.maximum(y, 0.0)
    hh_ref[...] = y
    m = y + 1e-7
    x = jnp.exp(m)
    mx = m * x
    t2_ref[0] = jnp.concatenate([x[:, :64], mx[:, :64]], axis=1)
    t2_ref[1] = jnp.concatenate([x[:, 64:], mx[:, 64:]], axis=1)

  return pl.pallas_call(
      body,
      grid=(_N // _BLK,),
      in_specs=[
          pl.BlockSpec((_BLK, _D), lambda i: (i, 0)),
          pl.BlockSpec((1, _D), lambda i: (0, 0)),
          pl.BlockSpec((1, _D), lambda i: (0, 0)),
      ],
      out_specs=[
          pl.BlockSpec((_BLK, _D), lambda i: (i, 0)),
          pl.BlockSpec((2, _BLK, _D), lambda i: (0, i, 0)),
      ],
      out_shape=[
          jax.ShapeDtypeStruct((_N, _D), jnp.float32),
          jax.ShapeDtypeStruct((2, _N, _D), jnp.float32),
      ],
  )(h, g.reshape(1, _D), b.reshape(1, _D))


def _mlp_stats(acc, hh, w1, b1):
  def body(acc_ref, hh_ref, w1_ref, b1_ref, t_ref, st_ref):
    i = pl.program_id(0)
    ex = jnp.concatenate([acc_ref[0, :, :64], acc_ref[1, :, :64]], axis=1)
    mex = jnp.concatenate([acc_ref[0, :, 64:], acc_ref[1, :, 64:]], axis=1)
    out = mex / (ex + 1e-16) + hh_ref[...]
    t = (jnp.dot(out, w1_ref[...], preferred_element_type=jnp.float32)
         + b1_ref[...])
    t_ref[...] = t

    @pl.when(i == 0)
    def _():
      st_ref[...] = jnp.zeros_like(st_ref)

    st_ref[0:1, :] += jnp.sum(t, axis=0, keepdims=True)
    st_ref[1:2, :] += jnp.sum(t * t, axis=0, keepdims=True)

  return pl.pallas_call(
      body,
      grid=(_N // _BLK,),
      in_specs=[
          pl.BlockSpec((2, _BLK, 128), lambda i: (0, i, 0)),
          pl.BlockSpec((_BLK, _D), lambda i: (i, 0)),
          pl.BlockSpec((_D, 2 * _D), lambda i: (0, 0)),
          pl.BlockSpec((1, 2 * _D), lambda i: (0, 0)),
      ],
      out_specs=[
          pl.BlockSpec((_BLK, 2 * _D), lambda i: (i, 0)),
          pl.BlockSpec((2, 2 * _D), lambda i: (0, 0)),
      ],
      out_shape=[
          jax.ShapeDtypeStruct((_N, 2 * _D), jnp.float32),
          jax.ShapeDtypeStruct((2, 2 * _D), jnp.float32),
      ],
  )(acc, hh, w1, b1.reshape(1, 2 * _D))


def _mlp_apply(t, st, h, g, b, w2, b2):
  def body(t_ref, st_ref, h_ref, g_ref, b_ref, w2_ref, b2_ref, o_ref):
    mu = st_ref[0:1, :] * (1.0 / _N)
    var = st_ref[1:2, :] * (1.0 / _N) - mu * mu
    tb = g_ref[...] * (t_ref[...] - mu) / jnp.sqrt(var + 1e-5) + b_ref[...]
    tb = jnp.maximum(tb, 0.0)
    o_ref[...] = (
        h_ref[...]
        + jnp.dot(tb, w2_ref[...], preferred_element_type=jnp.float32)
        + b2_ref[...])

  return pl.pallas_call(
      body,
      grid=(_N // _BLK,),
      in_specs=[
          pl.BlockSpec((_BLK, 2 * _D), lambda i: (i, 0)),
          pl.BlockSpec((2, 2 * _D), lambda i: (0, 0)),
          pl.BlockSpec((_BLK, _D), lambda i: (i, 0)),
          pl.BlockSpec((1, 2 * _D), lambda i: (0, 0)),
          pl.BlockSpec((1, 2 * _D), lambda i: (0, 0)),
          pl.BlockSpec((2 * _D, _D), lambda i: (0, 0)),
          pl.BlockSpec((1, _D), lambda i: (0, 0)),
      ],
      out_specs=pl.BlockSpec((_BLK, _D), lambda i: (i, 0)),
      out_shape=jax.ShapeDtypeStruct((_N, _D), jnp.float32),
  )(t, st, h, g.reshape(1, 2 * _D), b.reshape(1, 2 * _D), w2,
    b2.reshape(1, _D))


def kernel(x, edge_index, W_enc, b_enc, ln_gamma, ln_beta, W1, b1, bn_gamma,
           bn_beta, W2, b2, W_dec, b_dec):
  src3 = jnp.concatenate(
      [edge_index[0].reshape(_NS, _EPT),
       jnp.zeros((_NS, _PAD), jnp.int32)], axis=1).reshape(_NS, _NCHUNK, _W)
  dst3 = jnp.concatenate(
      [edge_index[1].reshape(_NS, _EPT),
       jnp.full((_NS, _PAD), _N, jnp.int32)], axis=1).reshape(_NS, _NCHUNK, _W)
  h = _encode(x, W_enc, b_enc)
  for l in range(4):
    hh, t2 = _ln_table(h, ln_gamma[l], ln_beta[l])
    acc = _edge_phase(t2.reshape(2 * _N, _D), src3, dst3)
    t, st = _mlp_stats(acc, hh, W1[l], b1[l])
    h = _mlp_apply(t, st, h, bn_gamma[l], bn_beta[l], W2[l], b2[l])
  return _encode(h, W_dec, b_dec)


# speedup vs baseline: 1.1459x; 1.1459x over previous
import functools

import jax
import jax.numpy as jnp
from jax import lax
from jax.experimental import pallas as pl
from jax.experimental.pallas import tpu as pltpu
from jax.experimental.pallas import tpu_sc as plsc

_N = 10000
_E = 320000
_D = 128
_NC = 2
_NS = 16
_W = 32
_NCHUNK = 632
_EPT = _E // _NS
_PAD = _NCHUNK * _W - _EPT
_RPT = 640
_ZB = 16


def _edge_phase(t2, sd3):
  mesh = plsc.VectorSubcoreMesh(core_axis_name="c", subcore_axis_name="s")

  @functools.partial(
      pl.kernel,
      out_type=jax.ShapeDtypeStruct((_NC, _N, 2 * 64), jnp.float32),
      mesh=mesh,
      scratch_types=[
          pltpu.VMEM((8, 2, _W), jnp.int32),
          pltpu.VMEM((_W, 128), jnp.float32),
          pltpu.VMEM((_W, 128), jnp.float32),
          pltpu.VMEM((_W, 128), jnp.float32),
          pltpu.VMEM((_W, 128), jnp.float32),
          pltpu.VMEM_SHARED((_N + 8, 128), jnp.float32),
          pltpu.SemaphoreType.DMA,
          pltpu.SemaphoreType.DMA,
          pltpu.SemaphoreType.DMA,
          pltpu.SemaphoreType.DMA,
          pltpu.SemaphoreType.DMA,
          pltpu.SemaphoreType.DMA,
          pltpu.SemaphoreType.DMA,
          pltpu.SemaphoreType.DMA,
          pltpu.SemaphoreType.DMA,
          pltpu.SemaphoreType.DMA,
          pltpu.SemaphoreType.DMA,
          pltpu.SemaphoreType.DMA,
          pltpu.SemaphoreType.DMA,
          pltpu.SemaphoreType.DMA,
          pltpu.SemaphoreType.DMA,
          pltpu.SemaphoreType.DMA,
      ],
  )
  def k(t2_hbm, sd_hbm, acc_hbm, isd, buf0, buf1, buf2,
        buf3, acc_sh, gsem0, gsem1, gsem2, gsem3, ssem0, ssem1, ssem2,
        ssem3, isem0, isem1, isem2, isem3, isem4, isem5, isem6, isem7):
    c = lax.axis_index("c")
    s = lax.axis_index("s")
    zeros16 = jnp.zeros((16,), jnp.float32)
    buf = (buf0, buf1, buf2, buf3)
    gsem = (gsem0, gsem1, gsem2, gsem3)
    ssem = (ssem0, ssem1, ssem2, ssem3)
    isem = (isem0, isem1, isem2, isem3, isem4, isem5, isem6, isem7)
    coff = c * _N

    def idx_start(q, slot):
      pltpu.async_copy(sd_hbm.at[s].at[q], isd.at[slot], isem[slot])

    def idx_wait_off(slot):
      pltpu.make_async_copy(sd_hbm.at[s].at[0], isd.at[slot],
                            isem[slot]).wait()
      for i in range(_W // 16):
        isd[slot, 0, pl.ds(i * 16, 16)] = (
            isd[slot, 0, pl.ds(i * 16, 16)] + coff)

    for q0 in range(4):
      idx_start(q0, q0)

    @pl.loop(0, _W)
    def _(e):
      for g in range(8):
        buf1[e, pl.ds(g * 16, 16)] = zeros16
        buf2[e, pl.ds(g * 16, 16)] = zeros16
        buf3[e, pl.ds(g * 16, 16)] = zeros16

    base_row = s * _RPT
    nrep = jnp.where(s == _NS - 1, (_N - (_NS - 1) * _RPT) // _ZB, _RPT // _ZB)

    @pl.loop(0, nrep)
    def _(j):
      pltpu.sync_copy(buf2.at[pl.ds(0, _ZB)],
                      acc_sh.at[pl.ds(base_row + j * _ZB, _ZB)])

    plsc.subcore_barrier()

    idx_wait_off(0)
    pltpu.async_copy(buf1, acc_sh.at[isd.at[0].at[1]], ssem1, add=True)
    pltpu.async_copy(buf2, acc_sh.at[isd.at[0].at[1]], ssem2, add=True)
    pltpu.async_copy(buf3, acc_sh.at[isd.at[0].at[1]], ssem3, add=True)
    pltpu.async_copy(t2_hbm.at[isd.at[0].at[0]], buf0, gsem0)

    def chunk_body(q, j, gather_next, idx_next):
      p = j % 4
      pn = (j + 1) % 4
      if gather_next:
        pltpu.make_async_copy(buf[pn], acc_sh.at[isd.at[j].at[1]],
                              ssem[pn]).wait()
        idx_wait_off((j + 1) % 8)
        pltpu.async_copy(t2_hbm.at[isd.at[(j + 1) % 8].at[0]], buf[pn], gsem[pn])
      if idx_next:
        idx_start(q + 4, (j + 4) % 8)
      pltpu.make_async_copy(t2_hbm.at[isd.at[j].at[0]], buf[p], gsem[p]).wait()
      pltpu.async_copy(buf[p], acc_sh.at[isd.at[j].at[1]], ssem[p], add=True)

    @pl.loop(0, _NCHUNK - 8, step=8)
    def _(kk):
      for j in range(8):
        chunk_body(kk + j, j, True, True)

    for j in range(8):
      q = _NCHUNK - 8 + j
      chunk_body(q, j, q + 1 < _NCHUNK, q + 4 < _NCHUNK)

    pltpu.make_async_copy(buf0, acc_sh.at[isd.at[0].at[1]], ssem0).wait()
    pltpu.make_async_copy(buf1, acc_sh.at[isd.at[1].at[1]], ssem1).wait()
    pltpu.make_async_copy(buf2, acc_sh.at[isd.at[2].at[1]], ssem2).wait()
    pltpu.make_async_copy(buf3, acc_sh.at[isd.at[3].at[1]], ssem3).wait()
    plsc.subcore_barrier()

    nrep2 = jnp.where(s == _NS - 1, (_N - (_NS - 1) * _RPT) // _ZB,
                      _RPT // _ZB)

    @pl.loop(0, nrep2)
    def _(j):
      r0 = base_row + j * _ZB
      pltpu.sync_copy(acc_sh.at[pl.ds(r0, _ZB)],
                      acc_hbm.at[c].at[pl.ds(r0, _ZB)])

  return k(t2, sd3)


_BLK = 400


def _encode(x, w, b):
  def body(x_ref, w_ref, b_ref, o_ref):
    o_ref[...] = (
        jnp.dot(x_ref[...], w_ref[...], preferred_element_type=jnp.float32)
        + b_ref[...])

  return pl.pallas_call(
      body,
      grid=(_N // _BLK,),
      in_specs=[
          pl.BlockSpec((_BLK, _D), lambda i: (i, 0)),
          pl.BlockSpec((_D, _D), lambda i: (0, 0)),
          pl.BlockSpec((1, _D), lambda i: (0, 0)),
      ],
      out_specs=pl.BlockSpec((_BLK, _D), lambda i: (i, 0)),
      out_shape=jax.ShapeDtypeStruct((_N, _D), jnp.float32),
  )(x, w, b.reshape(1, _D))


def _ln_table(h, g, b):
  def body(h_ref, g_ref, b_ref, hh_ref, t2_ref):
    hv = h_ref[...]
    mu = jnp.mean(hv, axis=1, keepdims=True)
    d = hv - mu
    var = jnp.mean(d * d, axis=1, keepdims=True)
    y = g_ref[...] * d / jnp.sqrt(var + 1e-5) + b_ref[...]
    y = jnp.maximum(y, 0.0)
    hh_ref[...] = y
    m = y + 1e-7
    x = jnp.exp(m)
    mx = m * x
    t2_ref[0] = jnp.concatenate([x[:, :64], mx[:, :64]], axis=1)
    t2_ref[1] = jnp.concatenate([x[:, 64:], mx[:, 64:]], axis=1)

  return pl.pallas_call(
      body,
      grid=(_N // _BLK,),
      in_specs=[
          pl.BlockSpec((_BLK, _D), lambda i: (i, 0)),
          pl.BlockSpec((1, _D), lambda i: (0, 0)),
          pl.BlockSpec((1, _D), lambda i: (0, 0)),
      ],
      out_specs=[
          pl.BlockSpec((_BLK, _D), lambda i: (i, 0)),
          pl.BlockSpec((2, _BLK, _D), lambda i: (0, i, 0)),
      ],
      out_shape=[
          jax.ShapeDtypeStruct((_N, _D), jnp.float32),
          jax.ShapeDtypeStruct((2, _N, _D), jnp.float32),
      ],
  )(h, g.reshape(1, _D), b.reshape(1, _D))


def _mlp_stats(acc, hh, w1, b1):
  def body(acc_ref, hh_ref, w1_ref, b1_ref, t_ref, st_ref):
    i = pl.program_id(0)
    ex = jnp.concatenate([acc_ref[0, :, :64], acc_ref[1, :, :64]], axis=1)
    mex = jnp.concatenate([acc_ref[0, :, 64:], acc_ref[1, :, 64:]], axis=1)
    out = mex / (ex + 1e-16) + hh_ref[...]
    t = (jnp.dot(out, w1_ref[...], preferred_element_type=jnp.float32)
         + b1_ref[...])
    t_ref[...] = t

    @pl.when(i == 0)
    def _():
      st_ref[...] = jnp.zeros_like(st_ref)

    st_ref[0:1, :] += jnp.sum(t, axis=0, keepdims=True)
    st_ref[1:2, :] += jnp.sum(t * t, axis=0, keepdims=True)

  return pl.pallas_call(
      body,
      grid=(_N // _BLK,),
      in_specs=[
          pl.BlockSpec((2, _BLK, 128), lambda i: (0, i, 0)),
          pl.BlockSpec((_BLK, _D), lambda i: (i, 0)),
          pl.BlockSpec((_D, 2 * _D), lambda i: (0, 0)),
          pl.BlockSpec((1, 2 * _D), lambda i: (0, 0)),
      ],
      out_specs=[
          pl.BlockSpec((_BLK, 2 * _D), lambda i: (i, 0)),
          pl.BlockSpec((2, 2 * _D), lambda i: (0, 0)),
      ],
      out_shape=[
          jax.ShapeDtypeStruct((_N, 2 * _D), jnp.float32),
          jax.ShapeDtypeStruct((2, 2 * _D), jnp.float32),
      ],
  )(acc, hh, w1, b1.reshape(1, 2 * _D))


def _mlp_apply(t, st, h, g, b, w2, b2):
  def body(t_ref, st_ref, h_ref, g_ref, b_ref, w2_ref, b2_ref, o_ref):
    mu = st_ref[0:1, :] * (1.0 / _N)
    var = st_ref[1:2, :] * (1.0 / _N) - mu * mu
    tb = g_ref[...] * (t_ref[...] - mu) / jnp.sqrt(var + 1e-5) + b_ref[...]
    tb = jnp.maximum(tb, 0.0)
    o_ref[...] = (
        h_ref[...]
        + jnp.dot(tb, w2_ref[...], preferred_element_type=jnp.float32)
        + b2_ref[...])

  return pl.pallas_call(
      body,
      grid=(_N // _BLK,),
      in_specs=[
          pl.BlockSpec((_BLK, 2 * _D), lambda i: (i, 0)),
          pl.BlockSpec((2, 2 * _D), lambda i: (0, 0)),
          pl.BlockSpec((_BLK, _D), lambda i: (i, 0)),
          pl.BlockSpec((1, 2 * _D), lambda i: (0, 0)),
          pl.BlockSpec((1, 2 * _D), lambda i: (0, 0)),
          pl.BlockSpec((2 * _D, _D), lambda i: (0, 0)),
          pl.BlockSpec((1, _D), lambda i: (0, 0)),
      ],
      out_specs=pl.BlockSpec((_BLK, _D), lambda i: (i, 0)),
      out_shape=jax.ShapeDtypeStruct((_N, _D), jnp.float32),
  )(t, st, h, g.reshape(1, 2 * _D), b.reshape(1, 2 * _D), w2,
    b2.reshape(1, _D))


def kernel(x, edge_index, W_enc, b_enc, ln_gamma, ln_beta, W1, b1, bn_gamma,
           bn_beta, W2, b2, W_dec, b_dec):
  src3 = jnp.concatenate(
      [edge_index[0].reshape(_NS, _EPT),
       jnp.zeros((_NS, _PAD), jnp.int32)], axis=1).reshape(_NS, _NCHUNK, _W)
  dst3 = jnp.concatenate(
      [edge_index[1].reshape(_NS, _EPT),
       jnp.full((_NS, _PAD), _N, jnp.int32)], axis=1).reshape(_NS, _NCHUNK, _W)
  sd3 = jnp.stack([src3, dst3], axis=2)
  h = _encode(x, W_enc, b_enc)
  for l in range(4):
    hh, t2 = _ln_table(h, ln_gamma[l], ln_beta[l])
    acc = _edge_phase(t2.reshape(2 * _N, _D), sd3)
    t, st = _mlp_stats(acc, hh, W1[l], b1[l])
    h = _mlp_apply(t, st, h, bn_gamma[l], bn_beta[l], W2[l], b2[l])
  return _encode(h, W_dec, b_dec)


# speedup vs baseline: 1.1570x; 1.0096x over previous
import functools

import jax
import jax.numpy as jnp
from jax import lax
from jax.experimental import pallas as pl
from jax.experimental.pallas import tpu as pltpu
from jax.experimental.pallas import tpu_sc as plsc

_N = 10000
_E = 320000
_D = 128
_NC = 2
_NS = 16
_W = 32
_NCHUNK = 632
_EPT = _E // _NS
_PAD = _NCHUNK * _W - _EPT
_RPT = 640
_ZB = 16


def _edge_phase(t2, src3, dst3):
  mesh = plsc.VectorSubcoreMesh(core_axis_name="c", subcore_axis_name="s")

  @functools.partial(
      pl.kernel,
      out_type=jax.ShapeDtypeStruct((_NC, _N, 2 * 64), jnp.float32),
      mesh=mesh,
      scratch_types=[
          pltpu.VMEM((8, _W), jnp.int32),
          pltpu.VMEM((8, _W), jnp.int32),
          pltpu.VMEM((_W, 128), jnp.float32),
          pltpu.VMEM((_W, 128), jnp.float32),
          pltpu.VMEM((_W, 128), jnp.float32),
          pltpu.VMEM((_W, 128), jnp.float32),
          pltpu.VMEM_SHARED((_N + 8, 128), jnp.float32),
          pltpu.SemaphoreType.DMA,
          pltpu.SemaphoreType.DMA,
          pltpu.SemaphoreType.DMA,
          pltpu.SemaphoreType.DMA,
          pltpu.SemaphoreType.DMA,
          pltpu.SemaphoreType.DMA,
          pltpu.SemaphoreType.DMA,
          pltpu.SemaphoreType.DMA,
          pltpu.SemaphoreType.DMA,
          pltpu.SemaphoreType.DMA,
          pltpu.SemaphoreType.DMA,
          pltpu.SemaphoreType.DMA,
          pltpu.SemaphoreType.DMA,
          pltpu.SemaphoreType.DMA,
          pltpu.SemaphoreType.DMA,
          pltpu.SemaphoreType.DMA,
      ],
  )
  def k(t2_hbm, src_hbm, dst_hbm, acc_hbm, isrc, idst, buf0, buf1, buf2,
        buf3, acc_sh, gsem0, gsem1, gsem2, gsem3, ssem0, ssem1, ssem2,
        ssem3, isem0, isem1, isem2, isem3, isem4, isem5, isem6, isem7):
    c = lax.axis_index("c")
    s = lax.axis_index("s")
    zeros16 = jnp.zeros((16,), jnp.float32)
    buf = (buf0, buf1, buf2, buf3)
    gsem = (gsem0, gsem1, gsem2, gsem3)
    ssem = (ssem0, ssem1, ssem2, ssem3)
    isem = (isem0, isem1, isem2, isem3, isem4, isem5, isem6, isem7)
    coff = c * _N

    def idx_start(q, slot):
      pltpu.async_copy(src_hbm.at[s].at[q], isrc.at[slot], isem[slot])
      pltpu.async_copy(dst_hbm.at[s].at[q], idst.at[slot], isem[slot])

    def idx_wait_off(slot):
      pltpu.make_async_copy(src_hbm.at[s].at[0], isrc.at[slot],
                            isem[slot]).wait()
      pltpu.make_async_copy(dst_hbm.at[s].at[0], idst.at[slot],
                            isem[slot]).wait()
      for i in range(_W // 16):
        isrc[slot, pl.ds(i * 16, 16)] = isrc[slot, pl.ds(i * 16, 16)] + coff

    for q0 in range(4):
      idx_start(q0, q0)

    @pl.loop(0, _W)
    def _(e):
      for g in range(8):
        buf1[e, pl.ds(g * 16, 16)] = zeros16
        buf2[e, pl.ds(g * 16, 16)] = zeros16
        buf3[e, pl.ds(g * 16, 16)] = zeros16

    base_row = s * _RPT
    nrep = jnp.where(s == _NS - 1, (_N - (_NS - 1) * _RPT) // _ZB, _RPT // _ZB)

    @pl.loop(0, nrep)
    def _(j):
      pltpu.sync_copy(buf2.at[pl.ds(0, _ZB)],
                      acc_sh.at[pl.ds(base_row + j * _ZB, _ZB)])

    plsc.subcore_barrier()

    idx_wait_off(0)
    pltpu.async_copy(buf1, acc_sh.at[idst.at[0]], ssem1, add=True)
    pltpu.async_copy(buf2, acc_sh.at[idst.at[0]], ssem2, add=True)
    pltpu.async_copy(buf3, acc_sh.at[idst.at[0]], ssem3, add=True)
    pltpu.async_copy(t2_hbm.at[isrc.at[0]], buf0, gsem0)

    def chunk_body(q, j, gather_next, idx_next):
      p = j % 4
      pn = (j + 1) % 4
      if gather_next:
        pltpu.make_async_copy(buf[pn], acc_sh.at[idst.at[j]],
                              ssem[pn]).wait()
        idx_wait_off((j + 1) % 8)
        pltpu.async_copy(t2_hbm.at[isrc.at[(j + 1) % 8]], buf[pn], gsem[pn])
      if idx_next:
        idx_start(q + 4, (j + 4) % 8)
      pltpu.make_async_copy(t2_hbm.at[isrc.at[j]], buf[p], gsem[p]).wait()
      pltpu.async_copy(buf[p], acc_sh.at[idst.at[j]], ssem[p], add=True)

    @pl.loop(0, _NCHUNK - 8, step=8)
    def _(kk):
      for j in range(8):
        chunk_body(kk + j, j, True, True)

    for j in range(8):
      q = _NCHUNK - 8 + j
      chunk_body(q, j, q + 1 < _NCHUNK, q + 4 < _NCHUNK)

    pltpu.make_async_copy(buf0, acc_sh.at[idst.at[0]], ssem0).wait()
    pltpu.make_async_copy(buf1, acc_sh.at[idst.at[1]], ssem1).wait()
    pltpu.make_async_copy(buf2, acc_sh.at[idst.at[2]], ssem2).wait()
    pltpu.make_async_copy(buf3, acc_sh.at[idst.at[3]], ssem3).wait()
    plsc.subcore_barrier()

    nrep2 = jnp.where(s == _NS - 1, (_N - (_NS - 1) * _RPT) // _ZB,
                      _RPT // _ZB)

    @pl.loop(0, nrep2)
    def _(j):
      r0 = base_row + j * _ZB
      pltpu.sync_copy(acc_sh.at[pl.ds(r0, _ZB)],
                      acc_hbm.at[c].at[pl.ds(r0, _ZB)])

  return k(t2, src3, dst3)


_BLK = 400


def _encode(x, w, b):
  def body(x_ref, w_ref, b_ref, o_ref):
    o_ref[...] = (
        jnp.dot(x_ref[...], w_ref[...], preferred_element_type=jnp.float32)
        + b_ref[...])

  return pl.pallas_call(
      body,
      grid=(_N // _BLK,),
      in_specs=[
          pl.BlockSpec((_BLK, _D), lambda i: (i, 0)),
          pl.BlockSpec((_D, _D), lambda i: (0, 0)),
          pl.BlockSpec((1, _D), lambda i: (0, 0)),
      ],
      out_specs=pl.BlockSpec((_BLK, _D), lambda i: (i, 0)),
      out_shape=jax.ShapeDtypeStruct((_N, _D), jnp.float32),
  )(x, w, b.reshape(1, _D))


def _ln_table(h, g, b):
  def body(h_ref, g_ref, b_ref, hh_ref, t2_ref):
    hv = h_ref[...]
    mu = jnp.mean(hv, axis=1, keepdims=True)
    d = hv - mu
    var = jnp.mean(d * d, axis=1, keepdims=True)
    y = g_ref[...] * d / jnp.sqrt(var + 1e-5) + b_ref[...]
    y = jnp.maximum(y, 0.0)
    hh_ref[...] = y
    m = y + 1e-7
    x = jnp.exp(m)
    mx = m * x
    t2_ref[0] = jnp.concatenate([x[:, :64], mx[:, :64]], axis=1)
    t2_ref[1] = jnp.concatenate([x[:, 64:], mx[:, 64:]], axis=1)

  return pl.pallas_call(
      body,
      grid=(_N // _BLK,),
      in_specs=[
          pl.BlockSpec((_BLK, _D), lambda i: (i, 0)),
          pl.BlockSpec((1, _D), lambda i: (0, 0)),
          pl.BlockSpec((1, _D), lambda i: (0, 0)),
      ],
      out_specs=[
          pl.BlockSpec((_BLK, _D), lambda i: (i, 0)),
          pl.BlockSpec((2, _BLK, _D), lambda i: (0, i, 0)),
      ],
      out_shape=[
          jax.ShapeDtypeStruct((_N, _D), jnp.float32),
          jax.ShapeDtypeStruct((2, _N, _D), jnp.float32),
      ],
  )(h, g.reshape(1, _D), b.reshape(1, _D))


def _mlp_stats(acc, hh, w1, b1):
  def body(acc_ref, hh_ref, w1_ref, b1_ref, t_ref, st_ref):
    i = pl.program_id(0)
    ex = jnp.concatenate([acc_ref[0, :, :64], acc_ref[1, :, :64]], axis=1)
    mex = jnp.concatenate([acc_ref[0, :, 64:], acc_ref[1, :, 64:]], axis=1)
    out = mex / (ex + 1e-16) + hh_ref[...]
    t = (jnp.dot(out, w1_ref[...], preferred_element_type=jnp.float32)
         + b1_ref[...])
    t_ref[...] = t

    @pl.when(i == 0)
    def _():
      st_ref[...] = jnp.zeros_like(st_ref)

    st_ref[0:1, :] += jnp.sum(t, axis=0, keepdims=True)
    st_ref[1:2, :] += jnp.sum(t * t, axis=0, keepdims=True)

  return pl.pallas_call(
      body,
      grid=(_N // _BLK,),
      in_specs=[
          pl.BlockSpec((2, _BLK, 128), lambda i: (0, i, 0)),
          pl.BlockSpec((_BLK, _D), lambda i: (i, 0)),
          pl.BlockSpec((_D, 2 * _D), lambda i: (0, 0)),
          pl.BlockSpec((1, 2 * _D), lambda i: (0, 0)),
      ],
      out_specs=[
          pl.BlockSpec((_BLK, 2 * _D), lambda i: (i, 0)),
          pl.BlockSpec((2, 2 * _D), lambda i: (0, 0)),
      ],
      out_shape=[
          jax.ShapeDtypeStruct((_N, 2 * _D), jnp.float32),
          jax.ShapeDtypeStruct((2, 2 * _D), jnp.float32),
      ],
  )(acc, hh, w1, b1.reshape(1, 2 * _D))


def _mlp_apply(t, st, h, g, b, w2, b2):
  def body(t_ref, st_ref, h_ref, g_ref, b_ref, w2_ref, b2_ref, o_ref):
    mu = st_ref[0:1, :] * (1.0 / _N)
    var = st_ref[1:2, :] * (1.0 / _N) - mu * mu
    tb = g_ref[...] * (t_ref[...] - mu) / jnp.sqrt(var + 1e-5) + b_ref[...]
    tb = jnp.maximum(tb, 0.0)
    o_ref[...] = (
        h_ref[...]
        + jnp.dot(tb, w2_ref[...], preferred_element_type=jnp.float32)
        + b2_ref[...])

  return pl.pallas_call(
      body,
      grid=(_N // _BLK,),
      in_specs=[
          pl.BlockSpec((_BLK, 2 * _D), lambda i: (i, 0)),
          pl.BlockSpec((2, 2 * _D), lambda i: (0, 0)),
          pl.BlockSpec((_BLK, _D), lambda i: (i, 0)),
          pl.BlockSpec((1, 2 * _D), lambda i: (0, 0)),
          pl.BlockSpec((1, 2 * _D), lambda i: (0, 0)),
          pl.BlockSpec((2 * _D, _D), lambda i: (0, 0)),
          pl.BlockSpec((1, _D), lambda i: (0, 0)),
      ],
      out_specs=pl.BlockSpec((_BLK, _D), lambda i: (i, 0)),
      out_shape=jax.ShapeDtypeStruct((_N, _D), jnp.float32),
  )(t, st, h, g.reshape(1, 2 * _D), b.reshape(1, 2 * _D), w2,
    b2.reshape(1, _D))


def kernel(x, edge_index, W_enc, b_enc, ln_gamma, ln_beta, W1, b1, bn_gamma,
           bn_beta, W2, b2, W_dec, b_dec):
  src3 = jnp.concatenate(
      [edge_index[0].reshape(_NS, _EPT),
       jnp.zeros((_NS, _PAD), jnp.int32)], axis=1).reshape(_NS, _NCHUNK, _W)
  dst3 = jnp.concatenate(
      [edge_index[1].reshape(_NS, _EPT),
       jnp.full((_NS, _PAD), _N, jnp.int32)], axis=1).reshape(_NS, _NCHUNK, _W)
  h = _encode(x, W_enc, b_enc)
  for l in range(4):
    hh, t2 = _ln_table(h, ln_gamma[l], ln_beta[l])
    acc = _edge_phase(t2.reshape(2 * _N, _D), src3, dst3)
    t, st = _mlp_stats(acc, hh, W1[l], b1[l])
    h = _mlp_apply(t, st, h, bn_gamma[l], bn_beta[l], W2[l], b2[l])
  return _encode(h, W_dec, b_dec)
